# baseline (speedup 1.0000x reference)
import functools

import jax
import jax.numpy as jnp
from jax import lax
from jax.experimental import pallas as pl
from jax.experimental.pallas import tpu as pltpu
from jax.experimental.pallas import tpu_sc as plsc

F_E = 16
F_X = 128
F_U = 32
F_OUT = 32
N = 10000
E = 320000
BG = 64
NSTEPS = 2

NC = 2
NS = 16
NW = NC * NS
EW = E // NW
CH = 80
NCHUNK = EW // CH

NT = 1000
NGRID = N // NT
TE = 8000
EGRID = E // TE
NSPLIT = 2

_f32 = jnp.float32


def _relu(v):
    return jnp.maximum(v, 0.0)


def _node_pre_body(x_ref, oh_ref, uo_ref, us_ref, w1xa_ref, w1xb_ref,
                   w1u_ref, b1_ref, p_ref, r_ref):
    t1 = uo_ref[...] @ w1xb_ref[...]
    t2 = us_ref[...] @ w1u_ref[...]
    p = x_ref[...] @ w1xa_ref[...] + oh_ref[...] @ t1
    p_ref[...] = p
    r_ref[...] = -p + oh_ref[...] @ t2 + b1_ref[...]


def _node_pre(x, oh, uo, us, w1xa, w1xb, w1u, b1, interpret=False):
    full = lambda s: pl.BlockSpec(s, lambda i: (0, 0))
    return pl.pallas_call(
        _node_pre_body,
        grid=(NGRID,),
        in_specs=[
            pl.BlockSpec((NT, F_X), lambda i: (i, 0)),
            pl.BlockSpec((NT, BG), lambda i: (i, 0)),
            full((BG, F_U)), full((BG, F_U)),
            full((F_X, 128)), full((F_U, 128)), full((F_U, 128)),
            full((1, 128)),
        ],
        out_specs=[pl.BlockSpec((NT, 128), lambda i: (i, 0))] * 2,
        out_shape=[jax.ShapeDtypeStruct((N, 128), _f32)] * 2,
        interpret=interpret,
    )(x, oh, uo, us, w1xa, w1xb, w1u, b1)


def _edge_mlp_body(g_ref, e_ref, w1e_ref, w2_ref, b2_ref,
                   w3p_ref, b3p_ref, out_ref):
    pre1 = g_ref[...] + e_ref[...] @ w1e_ref[...]
    h1 = _relu(pre1).astype(jnp.bfloat16)
    h1w2 = jnp.dot(h1, w2_ref[...].astype(jnp.bfloat16),
                   preferred_element_type=_f32)
    h2 = _relu(h1w2 + b2_ref[...]).astype(jnp.bfloat16)
    out_ref[...] = jnp.dot(h2, w3p_ref[...].astype(jnp.bfloat16),
                           preferred_element_type=_f32) + b3p_ref[...]


def _edge_mlp(g, e32, w1e32, w2, b2, w3p, b3p, interpret=False):
    w = w3p.shape[1]
    rows = g.shape[0]
    full = lambda s: pl.BlockSpec(s, lambda i: (0, 0))
    return pl.pallas_call(
        _edge_mlp_body,
        grid=(rows // TE,),
        in_specs=[
            pl.BlockSpec((TE, 128), lambda i: (i, 0)),
            pl.BlockSpec((TE, 32), lambda i: (i, 0)),
            full((32, 128)), full((128, 128)), full((1, 128)),
            full((128, w)), full((1, w)),
        ],
        out_specs=pl.BlockSpec((TE, w), lambda i: (i, 0)),
        out_shape=jax.ShapeDtypeStruct((rows, w), _f32),
        interpret=interpret,
    )(g, e32, w1e32, w2, b2, w3p, b3p)


def _node_mlp_body(nagg, *refs):
    x_ref = refs[0]
    agg_refs = refs[1:1 + nagg]
    (oh_ref, uo_ref, us_ref, n1a1_ref, n1b_ref, n1a2_ref, n1c_ref, nb1_ref,
     n2_ref, nb2_ref, n3_ref, nb3_ref, xn_ref) = refs[1 + nagg:]
    asum = agg_refs[0][...]
    for a in agg_refs[1:]:
        asum = asum + a[...]
    cnt = jnp.maximum(asum[:, 16:17], 1.0)
    aggm = asum[:, :16] / cnt
    tmp = uo_ref[...] @ n1a2_ref[...] + us_ref[...] @ n1c_ref[...] + nb1_ref[...]
    pre = x_ref[...] @ n1a1_ref[...] + aggm @ n1b_ref[...] + oh_ref[...] @ tmp
    h1 = _relu(pre)
    h2 = _relu(h1 @ n2_ref[...] + nb2_ref[...])
    xn_ref[...] = h2 @ n3_ref[...] + nb3_ref[...]


def _node_mlp(x, aggs, oh, uo, us, n1a1, n1b, n1a2, n1c, nb1,
              n2, nb2, n3, nb3, interpret=False):
    nagg = 2 * len(aggs)
    full = lambda s: pl.BlockSpec(s, lambda i: (0, 0))
    agg_specs = []
    agg_args = []
    for a in aggs:
        agg_specs.append(pl.BlockSpec((NT, 32), lambda i: (i, 0)))
        agg_specs.append(pl.BlockSpec((NT, 32), lambda i: (i + NGRID, 0)))
        agg_args += [a, a]
    return pl.pallas_call(
        functools.partial(_node_mlp_body, nagg),
        grid=(NGRID,),
        in_specs=[pl.BlockSpec((NT, F_X), lambda i: (i, 0))] + agg_specs + [
            pl.BlockSpec((NT, BG), lambda i: (i, 0)),
            full((BG, F_U)), full((BG, F_U)),
            full((F_X, 128)), full((F_E, 128)), full((F_U, 128)),
            full((F_U, 128)), full((1, 128)),
            full((128, 128)), full((1, 128)),
            full((128, F_X)), full((1, F_X)),
        ],
        out_specs=pl.BlockSpec((NT, F_X), lambda i: (i, 0)),
        out_shape=jax.ShapeDtypeStruct((N, F_X), _f32),
        interpret=interpret,
    )(x, *agg_args, oh, uo, us,
      n1a1, n1b, n1a2, n1c, nb1, n2, nb2, n3, nb3)


def _attn_glob_body(has_final, *refs):
    if has_final:
        (xn_ref, oh_ref, us_ref, attw_ref, g1_ref, gb1_ref, g2_ref, gb2_ref,
         g3_ref, gb3_ref, uoth_ref, f1_ref, fb1_ref, f2_ref, fb2_ref,
         f3_ref, fb3_ref, un_ref, out_ref) = refs
    else:
        (xn_ref, oh_ref, us_ref, attw_ref, g1_ref, gb1_ref, g2_ref, gb2_ref,
         g3_ref, gb3_ref, un_ref) = refs
    xn = xn_ref[...]
    oh = oh_ref[...]
    s = jnp.sum(xn * attw_ref[...], axis=1, keepdims=True)
    sm = jnp.max(jnp.where(oh > 0.0, s, -jnp.inf), axis=0, keepdims=True)
    sm = jnp.where(jnp.isfinite(sm), sm, 0.0)
    smb = jnp.sum(oh * sm, axis=1, keepdims=True)
    ex = jnp.exp(s - smb)
    den = jnp.sum(oh * ex, axis=0, keepdims=True) + 1e-9
    denb = jnp.sum(oh * den, axis=1, keepdims=True)
    w = ex / denb
    xa = lax.dot_general(oh, w * xn, (((0,), (0,)), ((), ())))
    hc = jnp.concatenate([xa, us_ref[...]], axis=1)
    h1 = _relu(hc @ g1_ref[...] + gb1_ref[...])
    h2 = _relu(h1 @ g2_ref[...] + gb2_ref[...])
    un = h2 @ g3_ref[...] + gb3_ref[...]
    un_ref[...] = un
    if has_final:
        fc = jnp.concatenate([uoth_ref[...], un], axis=1)
        fh1 = _relu(fc @ f1_ref[...] + fb1_ref[...])
        fh2 = _relu(fh1 @ f2_ref[...] + fb2_ref[...])
        out_ref[...] = fh2 @ f3_ref[...] + fb3_ref[...]


def _attn_glob(xn, oh, us, attw, gw, final_args=None, interpret=False):
    g1, gb1, g2, gb2, g3, gb3 = gw
    has_final = final_args is not None
    args = [xn, oh, us, attw, g1, gb1, g2, gb2, g3, gb3]
    out_shape = [jax.ShapeDtypeStruct((BG, F_U), _f32)]
    if has_final:
        uoth, f1, fb1, f2, fb2, f3, fb3 = final_args
        args += [uoth, f1, fb1, f2, fb2, f3, fb3]
        out_shape.append(jax.ShapeDtypeStruct((BG, F_OUT), _f32))
    res = pl.pallas_call(
        functools.partial(_attn_glob_body, has_final),
        out_shape=out_shape,
        interpret=interpret,
    )(*args)
    return res if has_final else (res[0], None)


def _sc_mesh():
    return plsc.VectorSubcoreMesh(core_axis_name="c", subcore_axis_name="s",
                                  num_cores=NC, num_subcores=NS)


NB = 2


def _pick_chunk(ew):
    for ch in (80, 40, 16, 8):
        if ew % ch == 0:
            return ch
    raise ValueError(ew)


@functools.cache
def _sc_gather_kernel(ew, ch):
    nchunk = ew // ch
    ngrp = (nchunk + NB - 1) // NB

    @functools.partial(
        pl.kernel,
        out_type=jax.ShapeDtypeStruct((ew * NW, 128), _f32),
        mesh=_sc_mesh(),
        scratch_types=[
            pltpu.VMEM((ew,), jnp.int32),
            pltpu.VMEM((ew,), jnp.int32),
            [pltpu.VMEM((ch, 128), _f32)] * NB,
            [pltpu.VMEM((ch, 128), _f32)] * NB,
            [pltpu.VMEM((ch, 128), _f32)] * NB,
            [pltpu.SemaphoreType.DMA] * NB,
            [pltpu.SemaphoreType.DMA] * NB,
            [pltpu.SemaphoreType.DMA] * NB,
        ],
    )
    def f(p_hbm, r_hbm, dst_hbm, src_hbm, g_hbm,
          idxd, idxs, bufp, bufr, obuf, semp, semr, semo):
        wid = lax.axis_index("s") * NC + lax.axis_index("c")
        base = wid * ew
        pltpu.sync_copy(dst_hbm.at[pl.ds(base, ew)], idxd)
        pltpu.sync_copy(src_hbm.at[pl.ds(base, ew)], idxs)

        def start_gather(c, b):
            pltpu.async_copy(p_hbm.at[idxd.at[pl.ds(c * ch, ch)]],
                             bufp[b], semp[b])
            pltpu.async_copy(r_hbm.at[idxs.at[pl.ds(c * ch, ch)]],
                             bufr[b], semr[b])

        for b in range(NB):
            start_gather(b, b)

        def group(g, carry):
            for b in range(NB):
                c = g * NB + b

                @pl.when(c < nchunk)
                def _process():
                    pltpu.make_async_copy(
                        p_hbm.at[idxd.at[pl.ds(0, ch)]], bufp[b], semp[b]
                    ).wait()
                    pltpu.make_async_copy(
                        r_hbm.at[idxs.at[pl.ds(0, ch)]], bufr[b], semr[b]
                    ).wait()

                    @pl.when(c >= NB)
                    def _drain_prev_store():
                        pltpu.make_async_copy(
                            obuf[b], g_hbm.at[pl.ds(0, ch)], semo[b]).wait()

                    @plsc.parallel_loop(0, ch, unroll=4)
                    def _add(row):
                        for k in range(8):
                            sl = pl.ds(k * 16, 16)
                            obuf[b][row, sl] = bufp[b][row, sl] + bufr[b][row, sl]

                    pltpu.async_copy(obuf[b], g_hbm.at[pl.ds(base + c * ch, ch)],
                                     semo[b])

                    @pl.when(c + NB < nchunk)
                    def _prefetch():
                        start_gather(c + NB, b)

            return carry

        lax.fori_loop(0, ngrp, group, 0)
        for b in range(NB):
            pltpu.make_async_copy(obuf[b], g_hbm.at[pl.ds(0, ch)],
                                  semo[b]).wait()

    return f


def _sc_gather(p, r, dst, src):
    ew = dst.shape[0] // NW
    return _sc_gather_kernel(ew, _pick_chunk(ew))(p, r, dst, src)


_NZ = 1000


NBS = 4


@functools.cache
def _sc_scatter_kernel(w, ew, ch):
    nchunk = ew // ch
    ngrps = (nchunk + NBS - 1) // NBS

    @functools.partial(
        pl.kernel,
        out_type=jax.ShapeDtypeStruct((2 * N, w), _f32),
        mesh=_sc_mesh(),
        scratch_types=[
            [pltpu.VMEM((ch,), jnp.int32)] * NBS,
            [pltpu.VMEM((ch, w), _f32)] * NBS,
            pltpu.VMEM_SHARED((N, w), _f32),
            [pltpu.SemaphoreType.DMA] * NBS,
            [pltpu.SemaphoreType.DMA] * NBS,
        ],
    )
    def f(en_hbm, dst_hbm, zeros_hbm, out_hbm, idxd, buf, table,
          semi, semb):
        cid = lax.axis_index("c")
        sid = lax.axis_index("s")
        wid = sid * NC + cid
        base = wid * ew
        row0 = sid * _NZ

        @pl.when(sid < N // _NZ)
        def _zero():
            pltpu.sync_copy(zeros_hbm.at[pl.ds(row0, _NZ)],
                            table.at[pl.ds(row0, _NZ)])

        plsc.subcore_barrier()

        def start_load(c, b):
            pltpu.async_copy(dst_hbm.at[pl.ds(base + c * ch, ch)],
                             idxd[b], semi[b])
            pltpu.async_copy(en_hbm.at[pl.ds(base + c * ch, ch)],
                             buf[b], semb[b])

        for b in range(NBS):
            start_load(b, b)

        def group(g, carry):
            for b in range(NBS):
                c = g * NBS + b

                @pl.when(c < nchunk)
                def _process():
                    pltpu.make_async_copy(
                        dst_hbm.at[pl.ds(0, ch)], idxd[b], semi[b]).wait()
                    pltpu.make_async_copy(
                        en_hbm.at[pl.ds(0, ch)], buf[b], semb[b]).wait()
                    pltpu.sync_copy(buf[b], table.at[idxd[b]], add=True)

                    @pl.when(c + NBS < nchunk)
                    def _prefetch():
                        start_load(c + NBS, b)

            return carry

        lax.fori_loop(0, ngrps, group, 0)
        plsc.subcore_barrier()

        @pl.when(sid < N // _NZ)
        def _copy_out():
            pltpu.sync_copy(table.at[pl.ds(row0, _NZ)],
                            out_hbm.at[pl.ds(cid * N + row0, _NZ)])

    return f


def _sc_scatter(en, dst, zeros_tab):
    ew = dst.shape[0] // NW
    return _sc_scatter_kernel(en.shape[1], ew, _pick_chunk(ew))(
        en, dst, zeros_tab)


def _split_edge_w(edge_mlp):
    (W1, b1), (W2, b2), (W3, b3) = edge_mlp
    w1e32 = jnp.zeros((32, 128), _f32).at[:F_E].set(W1[:F_E])
    w1xa = W1[F_E:F_E + F_X]
    w1xb = W1[F_E + F_X:F_E + F_X + F_U]
    w1u = W1[F_E + F_X + F_U:]
    w3p = jnp.zeros((128, 32), _f32).at[:, :F_E].set(W3)
    b3p = jnp.zeros((1, 32), _f32).at[0, :F_E].set(b3).at[0, F_E].set(1.0)
    return (w1xa, w1xb, w1u, b1.reshape(1, 128), w1e32,
            W2, b2.reshape(1, 128), (w3p, b3p), (W3, b3.reshape(1, F_E)))


def _split_node_w(node_mlp):
    (N1, nb1), (N2, nb2), (N3, nb3) = node_mlp
    n1a1 = N1[:F_X]
    n1a2 = N1[F_X:F_X + F_U]
    n1b = N1[F_X + F_U:F_X + F_U + F_E]
    n1c = N1[F_X + F_U + F_E:]
    return (n1a1, n1b, n1a2, n1c, nb1.reshape(1, 128),
            N2, nb2.reshape(1, 128), N3, nb3.reshape(1, F_X))


def _gnn_step(x, es, u_self, u_other, oh, srcs, dsts, zeros_tab,
              ew, nw, attw_row, gw, final_args, interpret=False):
    (w1xa, w1xb, w1u, b1r, w1e32, W2, b2r, (w3p, b3p), _) = ew
    p, r = _node_pre(x, oh, u_other, u_self, w1xa, w1xb, w1u, b1r,
                     interpret=interpret)
    gs = [_sc_gather(p, r, d_, s_) for s_, d_ in zip(srcs, dsts)]
    es_new = [_edge_mlp(g, e32, w1e32, W2, b2r, w3p, b3p, interpret=interpret)
              for g, e32 in zip(gs, es)]
    aggs = [_sc_scatter(en, d_, zeros_tab)
            for en, d_ in zip(es_new, dsts)]
    xn = _node_mlp(x, aggs, oh, u_other, u_self, *nw, interpret=interpret)
    un, out = _attn_glob(xn, oh, u_self, attw_row, gw, final_args=final_args,
                         interpret=interpret)
    return xn, es_new, un, out


def kernel(x1, edge_index1, e1, u1, batch1, x2, edge_index2, e2, u2, batch2,
           edge_mlp, node_mlp, att_w, glob_mlp, final_mlp):
    eh = E // NSPLIT
    halves = lambda a: tuple(a[i * eh:(i + 1) * eh] for i in range(NSPLIT))
    srcs1, dsts1 = halves(edge_index1[0]), halves(edge_index1[1])
    srcs2, dsts2 = halves(edge_index2[0]), halves(edge_index2[1])
    oh1 = (batch1[:, None] == jnp.arange(BG, dtype=jnp.int32)[None, :]).astype(_f32)
    oh2 = (batch2[:, None] == jnp.arange(BG, dtype=jnp.int32)[None, :]).astype(_f32)
    es1 = halves(jnp.pad(e1, ((0, 0), (0, 32 - F_E))))
    es2 = halves(jnp.pad(e2, ((0, 0), (0, 32 - F_E))))
    zeros32 = jnp.zeros((N, 32), _f32)

    ew = _split_edge_w(edge_mlp)
    nw = _split_node_w(node_mlp)
    attw_row = att_w.reshape(1, F_X)
    gw = (glob_mlp[0][0], glob_mlp[0][1].reshape(1, 128),
          glob_mlp[1][0], glob_mlp[1][1].reshape(1, 128),
          glob_mlp[2][0], glob_mlp[2][1].reshape(1, F_U))
    fw = (final_mlp[0][0], final_mlp[0][1].reshape(1, 128),
          final_mlp[1][0], final_mlp[1][1].reshape(1, 128),
          final_mlp[2][0], final_mlp[2][1].reshape(1, F_OUT))

    outs = []
    for _ in range(NSTEPS):
        x1, es1, u1, _ = _gnn_step(
            x1, es1, u1, u2, oh1, srcs1, dsts1, zeros32,
            ew, nw, attw_row, gw, None)
        x2, es2, u2, out = _gnn_step(
            x2, es2, u2, u1, oh2, srcs2, dsts2, zeros32,
            ew, nw, attw_row, gw, (u1,) + fw)
        outs.append(out)
    return tuple(outs)


# speedup vs baseline: 1.0285x; 1.0285x over previous
import functools

import jax
import jax.numpy as jnp
from jax import lax
from jax.experimental import pallas as pl
from jax.experimental.pallas import tpu as pltpu
from jax.experimental.pallas import tpu_sc as plsc

F_E = 16
F_X = 128
F_U = 32
F_OUT = 32
N = 10000
E = 320000
BG = 64
NSTEPS = 2

NC = 2
NS = 16
NW = NC * NS
EW = E // NW
CH = 80
NCHUNK = EW // CH

NT = 1000
NGRID = N // NT
TE = 8000
EGRID = E // TE
NSPLIT = 2

_f32 = jnp.float32


def _relu(v):
    return jnp.maximum(v, 0.0)


def _node_pre_body(x_ref, oh_ref, uo_ref, us_ref, w1xa_ref, w1xb_ref,
                   w1u_ref, b1_ref, p_ref, r_ref):
    t1 = uo_ref[...] @ w1xb_ref[...]
    t2 = us_ref[...] @ w1u_ref[...]
    p = x_ref[...] @ w1xa_ref[...] + oh_ref[...] @ t1
    p_ref[...] = p
    r_ref[...] = -p + oh_ref[...] @ t2 + b1_ref[...]


def _node_pre(x, oh, uo, us, w1xa, w1xb, w1u, b1, interpret=False):
    full = lambda s: pl.BlockSpec(s, lambda i: (0, 0))
    return pl.pallas_call(
        _node_pre_body,
        grid=(NGRID,),
        in_specs=[
            pl.BlockSpec((NT, F_X), lambda i: (i, 0)),
            pl.BlockSpec((NT, BG), lambda i: (i, 0)),
            full((BG, F_U)), full((BG, F_U)),
            full((F_X, 128)), full((F_U, 128)), full((F_U, 128)),
            full((1, 128)),
        ],
        out_specs=[pl.BlockSpec((NT, 128), lambda i: (i, 0))] * 2,
        out_shape=[jax.ShapeDtypeStruct((N, 128), _f32)] * 2,
        interpret=interpret,
    )(x, oh, uo, us, w1xa, w1xb, w1u, b1)


def _edge_mlp_body(g_ref, e_ref, w1e_ref, w2_ref, b2_ref,
                   w3p_ref, b3p_ref, out_ref):
    pre1 = g_ref[...] + e_ref[...] @ w1e_ref[...]
    h1 = _relu(pre1).astype(jnp.bfloat16)
    h1w2 = jnp.dot(h1, w2_ref[...].astype(jnp.bfloat16),
                   preferred_element_type=_f32)
    h2 = _relu(h1w2 + b2_ref[...]).astype(jnp.bfloat16)
    out_ref[...] = jnp.dot(h2, w3p_ref[...].astype(jnp.bfloat16),
                           preferred_element_type=_f32) + b3p_ref[...]


def _edge_mlp(g, e32, w1e32, w2, b2, w3p, b3p, interpret=False):
    w = w3p.shape[1]
    rows = g.shape[0]
    full = lambda s: pl.BlockSpec(s, lambda i: (0, 0))
    return pl.pallas_call(
        _edge_mlp_body,
        grid=(rows // TE,),
        in_specs=[
            pl.BlockSpec((TE, 128), lambda i: (i, 0)),
            pl.BlockSpec((TE, 32), lambda i: (i, 0)),
            full((32, 128)), full((128, 128)), full((1, 128)),
            full((128, w)), full((1, w)),
        ],
        out_specs=pl.BlockSpec((TE, w), lambda i: (i, 0)),
        out_shape=jax.ShapeDtypeStruct((rows, w), _f32),
        interpret=interpret,
    )(g, e32, w1e32, w2, b2, w3p, b3p)


def _node_mlp_body(nagg, *refs):
    x_ref = refs[0]
    agg_refs = refs[1:1 + nagg]
    (oh_ref, uo_ref, us_ref, n1a1_ref, n1b_ref, n1a2_ref, n1c_ref, nb1_ref,
     n2_ref, nb2_ref, n3_ref, nb3_ref, xn_ref) = refs[1 + nagg:]
    asum = agg_refs[0][...]
    for a in agg_refs[1:]:
        asum = asum + a[...]
    cnt = jnp.maximum(asum[:, 16:17], 1.0)
    aggm = asum[:, :16] / cnt
    tmp = uo_ref[...] @ n1a2_ref[...] + us_ref[...] @ n1c_ref[...] + nb1_ref[...]
    pre = x_ref[...] @ n1a1_ref[...] + aggm @ n1b_ref[...] + oh_ref[...] @ tmp
    h1 = _relu(pre)
    h2 = _relu(h1 @ n2_ref[...] + nb2_ref[...])
    xn_ref[...] = h2 @ n3_ref[...] + nb3_ref[...]


def _node_mlp(x, aggs, oh, uo, us, n1a1, n1b, n1a2, n1c, nb1,
              n2, nb2, n3, nb3, interpret=False):
    nagg = 2 * len(aggs)
    full = lambda s: pl.BlockSpec(s, lambda i: (0, 0))
    agg_specs = []
    agg_args = []
    for a in aggs:
        agg_specs.append(pl.BlockSpec((NT, 32), lambda i: (i, 0)))
        agg_specs.append(pl.BlockSpec((NT, 32), lambda i: (i + NGRID, 0)))
        agg_args += [a, a]
    return pl.pallas_call(
        functools.partial(_node_mlp_body, nagg),
        grid=(NGRID,),
        in_specs=[pl.BlockSpec((NT, F_X), lambda i: (i, 0))] + agg_specs + [
            pl.BlockSpec((NT, BG), lambda i: (i, 0)),
            full((BG, F_U)), full((BG, F_U)),
            full((F_X, 128)), full((F_E, 128)), full((F_U, 128)),
            full((F_U, 128)), full((1, 128)),
            full((128, 128)), full((1, 128)),
            full((128, F_X)), full((1, F_X)),
        ],
        out_specs=pl.BlockSpec((NT, F_X), lambda i: (i, 0)),
        out_shape=jax.ShapeDtypeStruct((N, F_X), _f32),
        interpret=interpret,
    )(x, *agg_args, oh, uo, us,
      n1a1, n1b, n1a2, n1c, nb1, n2, nb2, n3, nb3)


def _attn_glob_body(has_final, *refs):
    if has_final:
        (xn_ref, oh_ref, us_ref, attw_ref, g1_ref, gb1_ref, g2_ref, gb2_ref,
         g3_ref, gb3_ref, uoth_ref, f1_ref, fb1_ref, f2_ref, fb2_ref,
         f3_ref, fb3_ref, un_ref, out_ref) = refs
    else:
        (xn_ref, oh_ref, us_ref, attw_ref, g1_ref, gb1_ref, g2_ref, gb2_ref,
         g3_ref, gb3_ref, un_ref) = refs
    xn = xn_ref[...]
    oh = oh_ref[...]
    s = jnp.sum(xn * attw_ref[...], axis=1, keepdims=True)
    sm = jnp.max(jnp.where(oh > 0.0, s, -jnp.inf), axis=0, keepdims=True)
    sm = jnp.where(jnp.isfinite(sm), sm, 0.0)
    smb = jnp.sum(oh * sm, axis=1, keepdims=True)
    ex = jnp.exp(s - smb)
    den = jnp.sum(oh * ex, axis=0, keepdims=True) + 1e-9
    denb = jnp.sum(oh * den, axis=1, keepdims=True)
    w = ex / denb
    xa = lax.dot_general(oh, w * xn, (((0,), (0,)), ((), ())))
    hc = jnp.concatenate([xa, us_ref[...]], axis=1)
    h1 = _relu(hc @ g1_ref[...] + gb1_ref[...])
    h2 = _relu(h1 @ g2_ref[...] + gb2_ref[...])
    un = h2 @ g3_ref[...] + gb3_ref[...]
    un_ref[...] = un
    if has_final:
        fc = jnp.concatenate([uoth_ref[...], un], axis=1)
        fh1 = _relu(fc @ f1_ref[...] + fb1_ref[...])
        fh2 = _relu(fh1 @ f2_ref[...] + fb2_ref[...])
        out_ref[...] = fh2 @ f3_ref[...] + fb3_ref[...]


def _attn_glob(xn, oh, us, attw, gw, final_args=None, interpret=False):
    g1, gb1, g2, gb2, g3, gb3 = gw
    has_final = final_args is not None
    args = [xn, oh, us, attw, g1, gb1, g2, gb2, g3, gb3]
    out_shape = [jax.ShapeDtypeStruct((BG, F_U), _f32)]
    if has_final:
        uoth, f1, fb1, f2, fb2, f3, fb3 = final_args
        args += [uoth, f1, fb1, f2, fb2, f3, fb3]
        out_shape.append(jax.ShapeDtypeStruct((BG, F_OUT), _f32))
    res = pl.pallas_call(
        functools.partial(_attn_glob_body, has_final),
        out_shape=out_shape,
        interpret=interpret,
    )(*args)
    return res if has_final else (res[0], None)


def _sc_mesh():
    return plsc.VectorSubcoreMesh(core_axis_name="c", subcore_axis_name="s",
                                  num_cores=NC, num_subcores=NS)


NB = 2


def _pick_chunk(ew):
    for ch in (80, 40, 16, 8):
        if ew % ch == 0:
            return ch
    raise ValueError(ew)


@functools.cache
def _sc_gather_kernel(ew, ch):
    nchunk = ew // ch
    ngrp = (nchunk + NB - 1) // NB

    @functools.partial(
        pl.kernel,
        out_type=jax.ShapeDtypeStruct((ew * NW, 128), _f32),
        mesh=_sc_mesh(),
        scratch_types=[
            pltpu.VMEM((ew,), jnp.int32),
            pltpu.VMEM((ew,), jnp.int32),
            [pltpu.VMEM((ch, 128), _f32)] * NB,
            [pltpu.VMEM((ch, 128), _f32)] * NB,
            [pltpu.VMEM((ch, 128), _f32)] * NB,
            [pltpu.SemaphoreType.DMA] * NB,
            [pltpu.SemaphoreType.DMA] * NB,
            [pltpu.SemaphoreType.DMA] * NB,
        ],
    )
    def f(p_hbm, r_hbm, dst_hbm, src_hbm, g_hbm,
          idxd, idxs, bufp, bufr, obuf, semp, semr, semo):
        wid = lax.axis_index("s") * NC + lax.axis_index("c")
        base = wid * ew
        pltpu.sync_copy(dst_hbm.at[pl.ds(base, ew)], idxd)
        pltpu.sync_copy(src_hbm.at[pl.ds(base, ew)], idxs)

        def start_gather(c, b):
            pltpu.async_copy(p_hbm.at[idxd.at[pl.ds(c * ch, ch)]],
                             bufp[b], semp[b])
            pltpu.async_copy(r_hbm.at[idxs.at[pl.ds(c * ch, ch)]],
                             bufr[b], semr[b])

        for b in range(NB):
            start_gather(b, b)

        def group(g, carry):
            for b in range(NB):
                c = g * NB + b

                @pl.when(c < nchunk)
                def _process():
                    pltpu.make_async_copy(
                        p_hbm.at[idxd.at[pl.ds(0, ch)]], bufp[b], semp[b]
                    ).wait()
                    pltpu.make_async_copy(
                        r_hbm.at[idxs.at[pl.ds(0, ch)]], bufr[b], semr[b]
                    ).wait()

                    @pl.when(c >= NB)
                    def _drain_prev_store():
                        pltpu.make_async_copy(
                            obuf[b], g_hbm.at[pl.ds(0, ch)], semo[b]).wait()

                    @plsc.parallel_loop(0, ch, unroll=4)
                    def _add(row):
                        for k in range(8):
                            sl = pl.ds(k * 16, 16)
                            obuf[b][row, sl] = bufp[b][row, sl] + bufr[b][row, sl]

                    pltpu.async_copy(obuf[b], g_hbm.at[pl.ds(base + c * ch, ch)],
                                     semo[b])

                    @pl.when(c + NB < nchunk)
                    def _prefetch():
                        start_gather(c + NB, b)

            return carry

        lax.fori_loop(0, ngrp, group, 0)
        for b in range(NB):
            pltpu.make_async_copy(obuf[b], g_hbm.at[pl.ds(0, ch)],
                                  semo[b]).wait()

    return f


def _sc_gather(p, r, dst, src):
    ew = dst.shape[0] // NW
    return _sc_gather_kernel(ew, _pick_chunk(ew))(p, r, dst, src)


_NZ = 1000


NBS = 4


@functools.cache
def _sc_scatter_kernel(w, ew, ch):
    nchunk = ew // ch
    ngrps = (nchunk + NBS - 1) // NBS

    @functools.partial(
        pl.kernel,
        out_type=jax.ShapeDtypeStruct((2 * N, w), _f32),
        mesh=_sc_mesh(),
        scratch_types=[
            [pltpu.VMEM((ch,), jnp.int32)] * NBS,
            [pltpu.VMEM((ch, w), _f32)] * NBS,
            pltpu.VMEM_SHARED((N, w), _f32),
            [pltpu.SemaphoreType.DMA] * NBS,
            [pltpu.SemaphoreType.DMA] * NBS,
        ],
    )
    def f(en_hbm, dst_hbm, zeros_hbm, out_hbm, idxd, buf, table,
          semi, semb):
        cid = lax.axis_index("c")
        sid = lax.axis_index("s")
        wid = sid * NC + cid
        base = wid * ew
        row0 = sid * _NZ

        @pl.when(sid < N // _NZ)
        def _zero():
            pltpu.sync_copy(zeros_hbm.at[pl.ds(row0, _NZ)],
                            table.at[pl.ds(row0, _NZ)])

        plsc.subcore_barrier()

        def start_load(c, b):
            pltpu.async_copy(dst_hbm.at[pl.ds(base + c * ch, ch)],
                             idxd[b], semi[b])
            pltpu.async_copy(en_hbm.at[pl.ds(base + c * ch, ch)],
                             buf[b], semb[b])

        for b in range(NBS):
            start_load(b, b)

        def group(g, carry):
            for b in range(NBS):
                c = g * NBS + b

                @pl.when(c < nchunk)
                def _process():
                    pltpu.make_async_copy(
                        dst_hbm.at[pl.ds(0, ch)], idxd[b], semi[b]).wait()
                    pltpu.make_async_copy(
                        en_hbm.at[pl.ds(0, ch)], buf[b], semb[b]).wait()
                    pltpu.sync_copy(buf[b], table.at[idxd[b]], add=True)

                    @pl.when(c + NBS < nchunk)
                    def _prefetch():
                        start_load(c + NBS, b)

            return carry

        lax.fori_loop(0, ngrps, group, 0)
        plsc.subcore_barrier()

        @pl.when(sid < N // _NZ)
        def _copy_out():
            pltpu.sync_copy(table.at[pl.ds(row0, _NZ)],
                            out_hbm.at[pl.ds(cid * N + row0, _NZ)])

    return f


def _sc_scatter(en, dst, zeros_tab):
    ew = dst.shape[0] // NW
    return _sc_scatter_kernel(en.shape[1], ew, _pick_chunk(ew))(
        en, dst, zeros_tab)


def _split_edge_w(edge_mlp):
    (W1, b1), (W2, b2), (W3, b3) = edge_mlp
    w1e32 = jnp.zeros((32, 128), _f32).at[:F_E].set(W1[:F_E])
    w1xa = W1[F_E:F_E + F_X]
    w1xb = W1[F_E + F_X:F_E + F_X + F_U]
    w1u = W1[F_E + F_X + F_U:]
    w3p = jnp.zeros((128, 32), _f32).at[:, :F_E].set(W3)
    b3p = jnp.zeros((1, 32), _f32).at[0, :F_E].set(b3).at[0, F_E].set(1.0)
    return (w1xa, w1xb, w1u, b1.reshape(1, 128), w1e32,
            W2, b2.reshape(1, 128), (w3p, b3p), (W3, b3.reshape(1, F_E)))


def _split_node_w(node_mlp):
    (N1, nb1), (N2, nb2), (N3, nb3) = node_mlp
    n1a1 = N1[:F_X]
    n1a2 = N1[F_X:F_X + F_U]
    n1b = N1[F_X + F_U:F_X + F_U + F_E]
    n1c = N1[F_X + F_U + F_E:]
    return (n1a1, n1b, n1a2, n1c, nb1.reshape(1, 128),
            N2, nb2.reshape(1, 128), N3, nb3.reshape(1, F_X))


def _gnn_step(x, es, u_self, u_other, oh, srcs, dsts, zeros_tab,
              ew, nw, attw_row, gw, final_args, interpret=False):
    (w1xa, w1xb, w1u, b1r, w1e32, W2, b2r, (w3p, b3p), _) = ew
    p, r = _node_pre(x, oh, u_other, u_self, w1xa, w1xb, w1u, b1r,
                     interpret=interpret)
    gs = [_sc_gather(p, r, d_, s_) for s_, d_ in zip(srcs, dsts)]
    es_new = [_edge_mlp(g, e32, w1e32, W2, b2r, w3p, b3p, interpret=interpret)
              for g, e32 in zip(gs, es)]
    aggs = [_sc_scatter(en, d_, zeros_tab)
            for en, d_ in zip(es_new, dsts)]
    xn = _node_mlp(x, aggs, oh, u_other, u_self, *nw, interpret=interpret)
    un, out = _attn_glob(xn, oh, u_self, attw_row, gw, final_args=final_args,
                         interpret=interpret)
    return xn, es_new, un, out


def kernel(x1, edge_index1, e1, u1, batch1, x2, edge_index2, e2, u2, batch2,
           edge_mlp, node_mlp, att_w, glob_mlp, final_mlp):
    bounds = (0, 192000, E)
    halves = lambda a: tuple(a[bounds[i]:bounds[i + 1]]
                             for i in range(NSPLIT))
    srcs1, dsts1 = halves(edge_index1[0]), halves(edge_index1[1])
    srcs2, dsts2 = halves(edge_index2[0]), halves(edge_index2[1])
    oh1 = (batch1[:, None] == jnp.arange(BG, dtype=jnp.int32)[None, :]).astype(_f32)
    oh2 = (batch2[:, None] == jnp.arange(BG, dtype=jnp.int32)[None, :]).astype(_f32)
    es1 = halves(jnp.pad(e1, ((0, 0), (0, 32 - F_E))))
    es2 = halves(jnp.pad(e2, ((0, 0), (0, 32 - F_E))))
    zeros32 = jnp.zeros((N, 32), _f32)

    ew = _split_edge_w(edge_mlp)
    nw = _split_node_w(node_mlp)
    attw_row = att_w.reshape(1, F_X)
    gw = (glob_mlp[0][0], glob_mlp[0][1].reshape(1, 128),
          glob_mlp[1][0], glob_mlp[1][1].reshape(1, 128),
          glob_mlp[2][0], glob_mlp[2][1].reshape(1, F_U))
    fw = (final_mlp[0][0], final_mlp[0][1].reshape(1, 128),
          final_mlp[1][0], final_mlp[1][1].reshape(1, 128),
          final_mlp[2][0], final_mlp[2][1].reshape(1, F_OUT))

    outs = []
    for _ in range(NSTEPS):
        x1, es1, u1, _ = _gnn_step(
            x1, es1, u1, u2, oh1, srcs1, dsts1, zeros32,
            ew, nw, attw_row, gw, None)
        x2, es2, u2, out = _gnn_step(
            x2, es2, u2, u1, oh2, srcs2, dsts2, zeros32,
            ew, nw, attw_row, gw, (u1,) + fw)
        outs.append(out)
    return tuple(outs)


# speedup vs baseline: 1.0309x; 1.0023x over previous
import functools

import jax
import jax.numpy as jnp
from jax import lax
from jax.experimental import pallas as pl
from jax.experimental.pallas import tpu as pltpu
from jax.experimental.pallas import tpu_sc as plsc

F_E = 16
F_X = 128
F_U = 32
F_OUT = 32
N = 10000
E = 320000
BG = 64
NSTEPS = 2

NC = 2
NS = 16
NW = NC * NS
EW = E // NW
CH = 80
NCHUNK = EW // CH

NT = 1000
NGRID = N // NT
TE = 8000
EGRID = E // TE
NSPLIT = 2

_f32 = jnp.float32


def _relu(v):
    return jnp.maximum(v, 0.0)


def _node_pre_body(x_ref, oh_ref, uo_ref, us_ref, w1xa_ref, w1xb_ref,
                   w1u_ref, b1_ref, p_ref, r_ref):
    t1 = uo_ref[...] @ w1xb_ref[...]
    t2 = us_ref[...] @ w1u_ref[...]
    p = x_ref[...] @ w1xa_ref[...] + oh_ref[...] @ t1
    p_ref[...] = p
    r_ref[...] = -p + oh_ref[...] @ t2 + b1_ref[...]


def _node_pre(x, oh, uo, us, w1xa, w1xb, w1u, b1, interpret=False):
    full = lambda s: pl.BlockSpec(s, lambda i: (0, 0))
    return pl.pallas_call(
        _node_pre_body,
        grid=(NGRID,),
        in_specs=[
            pl.BlockSpec((NT, F_X), lambda i: (i, 0)),
            pl.BlockSpec((NT, BG), lambda i: (i, 0)),
            full((BG, F_U)), full((BG, F_U)),
            full((F_X, 128)), full((F_U, 128)), full((F_U, 128)),
            full((1, 128)),
        ],
        out_specs=[pl.BlockSpec((NT, 128), lambda i: (i, 0))] * 2,
        out_shape=[jax.ShapeDtypeStruct((N, 128), _f32)] * 2,
        interpret=interpret,
    )(x, oh, uo, us, w1xa, w1xb, w1u, b1)


def _edge_mlp_body(g_ref, e_ref, w1e_ref, w2_ref, b2_ref,
                   w3p_ref, b3p_ref, out_ref):
    pre1 = g_ref[...] + e_ref[...] @ w1e_ref[...]
    h1 = _relu(pre1).astype(jnp.bfloat16)
    h1w2 = jnp.dot(h1, w2_ref[...].astype(jnp.bfloat16),
                   preferred_element_type=_f32)
    h2 = _relu(h1w2 + b2_ref[...]).astype(jnp.bfloat16)
    out_ref[...] = jnp.dot(h2, w3p_ref[...].astype(jnp.bfloat16),
                           preferred_element_type=_f32) + b3p_ref[...]


def _edge_mlp(g, e32, w1e32, w2, b2, w3p, b3p, interpret=False):
    w = w3p.shape[1]
    rows = g.shape[0]
    full = lambda s: pl.BlockSpec(s, lambda i: (0, 0))
    return pl.pallas_call(
        _edge_mlp_body,
        grid=(rows // TE,),
        in_specs=[
            pl.BlockSpec((TE, 128), lambda i: (i, 0)),
            pl.BlockSpec((TE, 32), lambda i: (i, 0)),
            full((32, 128)), full((128, 128)), full((1, 128)),
            full((128, w)), full((1, w)),
        ],
        out_specs=pl.BlockSpec((TE, w), lambda i: (i, 0)),
        out_shape=jax.ShapeDtypeStruct((rows, w), _f32),
        interpret=interpret,
    )(g, e32, w1e32, w2, b2, w3p, b3p)


def _node_mlp_body(nagg, *refs):
    x_ref = refs[0]
    agg_refs = refs[1:1 + nagg]
    (oh_ref, uo_ref, us_ref, n1a1_ref, n1b_ref, n1a2_ref, n1c_ref, nb1_ref,
     n2_ref, nb2_ref, n3_ref, nb3_ref, xn_ref) = refs[1 + nagg:]
    asum = agg_refs[0][...]
    for a in agg_refs[1:]:
        asum = asum + a[...]
    cnt = jnp.maximum(asum[:, 16:17], 1.0)
    aggm = asum[:, :16] / cnt
    tmp = uo_ref[...] @ n1a2_ref[...] + us_ref[...] @ n1c_ref[...] + nb1_ref[...]
    pre = x_ref[...] @ n1a1_ref[...] + aggm @ n1b_ref[...] + oh_ref[...] @ tmp
    h1 = _relu(pre)
    h2 = _relu(h1 @ n2_ref[...] + nb2_ref[...])
    xn_ref[...] = h2 @ n3_ref[...] + nb3_ref[...]


def _node_mlp(x, aggs, oh, uo, us, n1a1, n1b, n1a2, n1c, nb1,
              n2, nb2, n3, nb3, interpret=False):
    nagg = 2 * len(aggs)
    full = lambda s: pl.BlockSpec(s, lambda i: (0, 0))
    agg_specs = []
    agg_args = []
    for a in aggs:
        agg_specs.append(pl.BlockSpec((NT, 32), lambda i: (i, 0)))
        agg_specs.append(pl.BlockSpec((NT, 32), lambda i: (i + NGRID, 0)))
        agg_args += [a, a]
    return pl.pallas_call(
        functools.partial(_node_mlp_body, nagg),
        grid=(NGRID,),
        in_specs=[pl.BlockSpec((NT, F_X), lambda i: (i, 0))] + agg_specs + [
            pl.BlockSpec((NT, BG), lambda i: (i, 0)),
            full((BG, F_U)), full((BG, F_U)),
            full((F_X, 128)), full((F_E, 128)), full((F_U, 128)),
            full((F_U, 128)), full((1, 128)),
            full((128, 128)), full((1, 128)),
            full((128, F_X)), full((1, F_X)),
        ],
        out_specs=pl.BlockSpec((NT, F_X), lambda i: (i, 0)),
        out_shape=jax.ShapeDtypeStruct((N, F_X), _f32),
        interpret=interpret,
    )(x, *agg_args, oh, uo, us,
      n1a1, n1b, n1a2, n1c, nb1, n2, nb2, n3, nb3)


def _attn_glob_body(has_final, *refs):
    if has_final:
        (xn_ref, oh_ref, us_ref, attw_ref, g1_ref, gb1_ref, g2_ref, gb2_ref,
         g3_ref, gb3_ref, uoth_ref, f1_ref, fb1_ref, f2_ref, fb2_ref,
         f3_ref, fb3_ref, un_ref, out_ref) = refs
    else:
        (xn_ref, oh_ref, us_ref, attw_ref, g1_ref, gb1_ref, g2_ref, gb2_ref,
         g3_ref, gb3_ref, un_ref) = refs
    xn = xn_ref[...]
    oh = oh_ref[...]
    s = jnp.sum(xn * attw_ref[...], axis=1, keepdims=True)
    sm = jnp.max(jnp.where(oh > 0.0, s, -jnp.inf), axis=0, keepdims=True)
    sm = jnp.where(jnp.isfinite(sm), sm, 0.0)
    smb = jnp.sum(oh * sm, axis=1, keepdims=True)
    ex = jnp.exp(s - smb)
    den = jnp.sum(oh * ex, axis=0, keepdims=True) + 1e-9
    denb = jnp.sum(oh * den, axis=1, keepdims=True)
    w = ex / denb
    xa = lax.dot_general(oh, w * xn, (((0,), (0,)), ((), ())))
    hc = jnp.concatenate([xa, us_ref[...]], axis=1)
    h1 = _relu(hc @ g1_ref[...] + gb1_ref[...])
    h2 = _relu(h1 @ g2_ref[...] + gb2_ref[...])
    un = h2 @ g3_ref[...] + gb3_ref[...]
    un_ref[...] = un
    if has_final:
        fc = jnp.concatenate([uoth_ref[...], un], axis=1)
        fh1 = _relu(fc @ f1_ref[...] + fb1_ref[...])
        fh2 = _relu(fh1 @ f2_ref[...] + fb2_ref[...])
        out_ref[...] = fh2 @ f3_ref[...] + fb3_ref[...]


def _attn_glob(xn, oh, us, attw, gw, final_args=None, interpret=False):
    g1, gb1, g2, gb2, g3, gb3 = gw
    has_final = final_args is not None
    args = [xn, oh, us, attw, g1, gb1, g2, gb2, g3, gb3]
    out_shape = [jax.ShapeDtypeStruct((BG, F_U), _f32)]
    if has_final:
        uoth, f1, fb1, f2, fb2, f3, fb3 = final_args
        args += [uoth, f1, fb1, f2, fb2, f3, fb3]
        out_shape.append(jax.ShapeDtypeStruct((BG, F_OUT), _f32))
    res = pl.pallas_call(
        functools.partial(_attn_glob_body, has_final),
        out_shape=out_shape,
        interpret=interpret,
    )(*args)
    return res if has_final else (res[0], None)


def _sc_mesh():
    return plsc.VectorSubcoreMesh(core_axis_name="c", subcore_axis_name="s",
                                  num_cores=NC, num_subcores=NS)


NBG = 3


def _pick_chunk(ew):
    for ch in (80, 40, 16, 8):
        if ew % ch == 0:
            return ch
    raise ValueError(ew)


@functools.cache
def _sc_gather_kernel(ew, ch):
    nchunk = ew // ch

    @functools.partial(
        pl.kernel,
        out_type=jax.ShapeDtypeStruct((ew * NW, 128), _f32),
        mesh=_sc_mesh(),
        scratch_types=[
            pltpu.VMEM((ew,), jnp.int32),
            pltpu.VMEM((ew,), jnp.int32),
            [pltpu.VMEM((ch, 128), _f32)] * NBG,
            [pltpu.VMEM((ch, 128), _f32)] * NBG,
            [pltpu.VMEM((ch, 128), _f32)] * NBG,
            [pltpu.SemaphoreType.DMA] * NBG,
            [pltpu.SemaphoreType.DMA] * NBG,
            [pltpu.SemaphoreType.DMA] * NBG,
        ],
    )
    def f(p_hbm, r_hbm, dst_hbm, src_hbm, g_hbm,
          idxd, idxs, bufp, bufr, obuf, semp, semr, semo):
        wid = lax.axis_index("s") * NC + lax.axis_index("c")
        base = wid * ew
        pltpu.sync_copy(dst_hbm.at[pl.ds(base, ew)], idxd)
        pltpu.sync_copy(src_hbm.at[pl.ds(base, ew)], idxs)

        def start_gather(c, b):
            pltpu.async_copy(p_hbm.at[idxd.at[pl.ds(c * ch, ch)]],
                             bufp[b], semp[b])
            pltpu.async_copy(r_hbm.at[idxs.at[pl.ds(c * ch, ch)]],
                             bufr[b], semr[b])

        for b in range(NBG):
            start_gather(b, b)

        def group(g, carry):
            for b in range(NBG):
                c = g * NBG + b

                @pl.when(c < nchunk)
                def _process():
                    pltpu.make_async_copy(
                        p_hbm.at[idxd.at[pl.ds(0, ch)]], bufp[b], semp[b]
                    ).wait()
                    pltpu.make_async_copy(
                        r_hbm.at[idxs.at[pl.ds(0, ch)]], bufr[b], semr[b]
                    ).wait()

                    @pl.when(c >= NBG)
                    def _drain_prev_store():
                        pltpu.make_async_copy(
                            obuf[b], g_hbm.at[pl.ds(0, ch)], semo[b]).wait()

                    @plsc.parallel_loop(0, ch, unroll=4)
                    def _add(row):
                        for k in range(8):
                            sl = pl.ds(k * 16, 16)
                            obuf[b][row, sl] = bufp[b][row, sl] + bufr[b][row, sl]

                    pltpu.async_copy(obuf[b], g_hbm.at[pl.ds(base + c * ch, ch)],
                                     semo[b])

                    @pl.when(c + NBG < nchunk)
                    def _prefetch():
                        start_gather(c + NBG, b)

            return carry

        lax.fori_loop(0, (nchunk + NBG - 1) // NBG, group, 0)
        for b in range(NBG):
            pltpu.make_async_copy(obuf[b], g_hbm.at[pl.ds(0, ch)],
                                  semo[b]).wait()

    return f


def _sc_gather(p, r, dst, src):
    ew = dst.shape[0] // NW
    return _sc_gather_kernel(ew, _pick_chunk(ew))(p, r, dst, src)


_NZ = 1000


NBS = 4


@functools.cache
def _sc_scatter_kernel(w, ew, ch):
    nchunk = ew // ch
    ngrps = (nchunk + NBS - 1) // NBS

    @functools.partial(
        pl.kernel,
        out_type=jax.ShapeDtypeStruct((2 * N, w), _f32),
        mesh=_sc_mesh(),
        scratch_types=[
            [pltpu.VMEM((ch,), jnp.int32)] * NBS,
            [pltpu.VMEM((ch, w), _f32)] * NBS,
            pltpu.VMEM_SHARED((N, w), _f32),
            [pltpu.SemaphoreType.DMA] * NBS,
            [pltpu.SemaphoreType.DMA] * NBS,
        ],
    )
    def f(en_hbm, dst_hbm, zeros_hbm, out_hbm, idxd, buf, table,
          semi, semb):
        cid = lax.axis_index("c")
        sid = lax.axis_index("s")
        wid = sid * NC + cid
        base = wid * ew
        row0 = sid * _NZ

        @pl.when(sid < N // _NZ)
        def _zero():
            pltpu.sync_copy(zeros_hbm.at[pl.ds(row0, _NZ)],
                            table.at[pl.ds(row0, _NZ)])

        plsc.subcore_barrier()

        def start_load(c, b):
            pltpu.async_copy(dst_hbm.at[pl.ds(base + c * ch, ch)],
                             idxd[b], semi[b])
            pltpu.async_copy(en_hbm.at[pl.ds(base + c * ch, ch)],
                             buf[b], semb[b])

        for b in range(NBS):
            start_load(b, b)

        def group(g, carry):
            for b in range(NBS):
                c = g * NBS + b

                @pl.when(c < nchunk)
                def _process():
                    pltpu.make_async_copy(
                        dst_hbm.at[pl.ds(0, ch)], idxd[b], semi[b]).wait()
                    pltpu.make_async_copy(
                        en_hbm.at[pl.ds(0, ch)], buf[b], semb[b]).wait()
                    pltpu.sync_copy(buf[b], table.at[idxd[b]], add=True)

                    @pl.when(c + NBS < nchunk)
                    def _prefetch():
                        start_load(c + NBS, b)

            return carry

        lax.fori_loop(0, ngrps, group, 0)
        plsc.subcore_barrier()

        @pl.when(sid < N // _NZ)
        def _copy_out():
            pltpu.sync_copy(table.at[pl.ds(row0, _NZ)],
                            out_hbm.at[pl.ds(cid * N + row0, _NZ)])

    return f


def _sc_scatter(en, dst, zeros_tab):
    ew = dst.shape[0] // NW
    return _sc_scatter_kernel(en.shape[1], ew, _pick_chunk(ew))(
        en, dst, zeros_tab)


def _split_edge_w(edge_mlp):
    (W1, b1), (W2, b2), (W3, b3) = edge_mlp
    w1e32 = jnp.zeros((32, 128), _f32).at[:F_E].set(W1[:F_E])
    w1xa = W1[F_E:F_E + F_X]
    w1xb = W1[F_E + F_X:F_E + F_X + F_U]
    w1u = W1[F_E + F_X + F_U:]
    w3p = jnp.zeros((128, 32), _f32).at[:, :F_E].set(W3)
    b3p = jnp.zeros((1, 32), _f32).at[0, :F_E].set(b3).at[0, F_E].set(1.0)
    return (w1xa, w1xb, w1u, b1.reshape(1, 128), w1e32,
            W2, b2.reshape(1, 128), (w3p, b3p), (W3, b3.reshape(1, F_E)))


def _split_node_w(node_mlp):
    (N1, nb1), (N2, nb2), (N3, nb3) = node_mlp
    n1a1 = N1[:F_X]
    n1a2 = N1[F_X:F_X + F_U]
    n1b = N1[F_X + F_U:F_X + F_U + F_E]
    n1c = N1[F_X + F_U + F_E:]
    return (n1a1, n1b, n1a2, n1c, nb1.reshape(1, 128),
            N2, nb2.reshape(1, 128), N3, nb3.reshape(1, F_X))


def _gnn_step(x, es, u_self, u_other, oh, srcs, dsts, zeros_tab,
              ew, nw, attw_row, gw, final_args, interpret=False):
    (w1xa, w1xb, w1u, b1r, w1e32, W2, b2r, (w3p, b3p), _) = ew
    p, r = _node_pre(x, oh, u_other, u_self, w1xa, w1xb, w1u, b1r,
                     interpret=interpret)
    gs = [_sc_gather(p, r, d_, s_) for s_, d_ in zip(srcs, dsts)]
    es_new = [_edge_mlp(g, e32, w1e32, W2, b2r, w3p, b3p, interpret=interpret)
              for g, e32 in zip(gs, es)]
    aggs = [_sc_scatter(en, d_, zeros_tab)
            for en, d_ in zip(es_new, dsts)]
    xn = _node_mlp(x, aggs, oh, u_other, u_self, *nw, interpret=interpret)
    un, out = _attn_glob(xn, oh, u_self, attw_row, gw, final_args=final_args,
                         interpret=interpret)
    return xn, es_new, un, out


def kernel(x1, edge_index1, e1, u1, batch1, x2, edge_index2, e2, u2, batch2,
           edge_mlp, node_mlp, att_w, glob_mlp, final_mlp):
    bounds = (0, 192000, E)
    halves = lambda a: tuple(a[bounds[i]:bounds[i + 1]]
                             for i in range(NSPLIT))
    srcs1, dsts1 = halves(edge_index1[0]), halves(edge_index1[1])
    srcs2, dsts2 = halves(edge_index2[0]), halves(edge_index2[1])
    oh1 = (batch1[:, None] == jnp.arange(BG, dtype=jnp.int32)[None, :]).astype(_f32)
    oh2 = (batch2[:, None] == jnp.arange(BG, dtype=jnp.int32)[None, :]).astype(_f32)
    es1 = halves(jnp.pad(e1, ((0, 0), (0, 32 - F_E))))
    es2 = halves(jnp.pad(e2, ((0, 0), (0, 32 - F_E))))
    zeros32 = jnp.zeros((N, 32), _f32)

    ew = _split_edge_w(edge_mlp)
    nw = _split_node_w(node_mlp)
    attw_row = att_w.reshape(1, F_X)
    gw = (glob_mlp[0][0], glob_mlp[0][1].reshape(1, 128),
          glob_mlp[1][0], glob_mlp[1][1].reshape(1, 128),
          glob_mlp[2][0], glob_mlp[2][1].reshape(1, F_U))
    fw = (final_mlp[0][0], final_mlp[0][1].reshape(1, 128),
          final_mlp[1][0], final_mlp[1][1].reshape(1, 128),
          final_mlp[2][0], final_mlp[2][1].reshape(1, F_OUT))

    outs = []
    for _ in range(NSTEPS):
        x1, es1, u1, _ = _gnn_step(
            x1, es1, u1, u2, oh1, srcs1, dsts1, zeros32,
            ew, nw, attw_row, gw, None)
        x2, es2, u2, out = _gnn_step(
            x2, es2, u2, u1, oh2, srcs2, dsts2, zeros32,
            ew, nw, attw_row, gw, (u1,) + fw)
        outs.append(out)
    return tuple(outs)
